# Initial kernel scaffold; baseline (speedup 1.0000x reference)
import jax
import jax.numpy as jnp
from jax.experimental import pallas as pl


def kernel(nf, pwr, pwr_feat, edge_index, node_level, Wp1, bp1, Wp2, bp2, Wr1, br1, Wr2, br2, Ws1, bs1, Ws2, bs2):
    raise NotImplementedError("write your pallas kernel here")


# speedup vs baseline: 22.2595x; 22.2595x over previous
import functools

import jax
import jax.numpy as jnp
from jax import lax
from jax.experimental import pallas as pl
from jax.experimental.pallas import tpu as pltpu
from jax.experimental.pallas import tpu_sc as plsc

F32 = jnp.float32
LANES = 16
NSUB = 16
NCORE = 2
NW = NCORE * NSUB
KB = 1024
RPB = KB // 128


def _leaky(x):
    return jnp.maximum(x, 0.2 * x)


def _init_body(nf_ref, wpt_ref, wpb_ref, bp1_ref, wrt_ref, br1_ref, nl_ref,
               a_ref, b_ref, c0_ref, em_ref):
    x = nf_ref[...]
    a_ref[...] = jnp.dot(x, wpt_ref[...], preferred_element_type=F32)
    b_ref[...] = jnp.dot(x, wpb_ref[...], preferred_element_type=F32) + bp1_ref[...]
    c0_ref[...] = jnp.dot(x, wrt_ref[...], preferred_element_type=F32) + br1_ref[...]
    em_ref[...] = jnp.where(nl_ref[...] == 1.0, 1.0, 0.0).astype(F32)


def _init_call(nf, wpt, wpb, bp1r, wrt, br1r, nl_f, bn):
    n = nf.shape[0]
    grid = (n // bn,)
    io = lambda i: (i, 0)
    w0 = lambda i: (0, 0)
    return pl.pallas_call(
        _init_body,
        grid=grid,
        in_specs=[
            pl.BlockSpec((bn, 128), io),
            pl.BlockSpec((128, 8), w0),
            pl.BlockSpec((128, 8), w0),
            pl.BlockSpec((1, 8), w0),
            pl.BlockSpec((128, 8), w0),
            pl.BlockSpec((1, 8), w0),
            pl.BlockSpec((bn, 1), io),
        ],
        out_specs=[
            pl.BlockSpec((bn, 8), io),
            pl.BlockSpec((bn, 8), io),
            pl.BlockSpec((bn, 8), io),
            pl.BlockSpec((bn, 1), io),
        ],
        out_shape=[
            jax.ShapeDtypeStruct((n, 8), F32),
            jax.ShapeDtypeStruct((n, 8), F32),
            jax.ShapeDtypeStruct((n, 8), F32),
            jax.ShapeDtypeStruct((n, 1), F32),
        ],
    )(nf, wpt, wpb, bp1r, wrt, br1r, nl_f)


def _edge_body(n2, nblk, s_hbm, d_hbm, srcr, dstr, w_hbm, z_hbm, out_hbm,
               acc, wbuf, idx_s, idx_d, sbuf, dbuf, obuf, sem1, sem2):
    c = lax.axis_index("c")
    s = lax.axis_index("s")
    wid = c * NSUB + s
    rpt = n2 // NSUB
    pltpu.sync_copy(z_hbm.at[pl.ds(s * rpt, rpt)], acc.at[pl.ds(s * rpt, rpt)])
    pltpu.sync_copy(w_hbm, wbuf)
    plsc.subcore_barrier()

    row_base = wid * nblk * RPB

    def group_body(g, car):
        rows = g * 16 + lax.iota(jnp.int32, 16)
        cidx = [jnp.full((16,), cc, jnp.int32) for cc in range(16)]
        scol = [plsc.load_gather(sbuf, [rows, cidx[cc]]) for cc in range(14)]
        dcol = [plsc.load_gather(dbuf, [rows, cidx[cc]]) for cc in range(9)]
        h = [_leaky(scol[f] + dcol[f]) for f in range(8)]
        e = []
        for j in range(9):
            acc_v = wbuf[72 + j]
            for f in range(8):
                acc_v = acc_v + h[f] * wbuf[f * 9 + j]
            e.append(acc_v)
        k = 1.0 / (1.0 + jnp.exp(-e[0]))
        m = dcol[8]
        km = k * m
        outc = [km * e[jj] for jj in range(1, 9)]
        outc.append(m)
        outc += [m * scol[8], m * scol[9]]
        outc += [m * scol[10 + t] for t in range(4)]
        outc.append(jnp.zeros((16,), F32))
        for cc in range(16):
            plsc.store_scatter(obuf, [rows, cidx[cc]], outc[cc])
        return car

    def blk_body(blk, car):
        r0 = row_base + blk * RPB
        pltpu.sync_copy(srcr.at[pl.ds(r0, RPB)], idx_s)
        pltpu.sync_copy(dstr.at[pl.ds(r0, RPB)], idx_d)
        descs = []
        for q in range(RPB):
            descs.append(pltpu.async_copy(
                s_hbm.at[idx_s.at[q]], sbuf.at[pl.ds(q * 128, 128)], sem1))
            descs.append(pltpu.async_copy(
                d_hbm.at[idx_d.at[q]], dbuf.at[pl.ds(q * 128, 128)], sem2))
        for dsc in descs:
            dsc.wait()
        car = lax.fori_loop(0, KB // 16, group_body, car)
        for q in range(RPB):
            pltpu.sync_copy(obuf.at[pl.ds(q * 128, 128)],
                            acc.at[idx_d.at[q]], add=True)
        return car

    lax.fori_loop(0, nblk, blk_body, jnp.int32(0))

    plsc.subcore_barrier()
    pltpu.sync_copy(acc.at[pl.ds(s * rpt, rpt)],
                    out_hbm.at[c, pl.ds(s * rpt, rpt)])


def _edge_call(s_tab, d_tab, srcr, dstr, w_b, zeros_n2, n2, nblk):
    mesh = plsc.VectorSubcoreMesh(core_axis_name="c", subcore_axis_name="s")
    body = functools.partial(_edge_body, n2, nblk)
    return pl.kernel(
        body,
        out_type=jax.ShapeDtypeStruct((NCORE, n2, 16), F32),
        mesh=mesh,
        compiler_params=pltpu.CompilerParams(needs_layout_passes=False,
                                             use_tc_tiling_on_sc=False),
        scratch_types=[
            pltpu.VMEM_SHARED((n2, 16), F32),
            pltpu.VMEM((96, 16), F32),
            pltpu.VMEM((RPB, 128), jnp.int32),
            pltpu.VMEM((RPB, 128), jnp.int32),
            pltpu.VMEM((KB, 16), F32),
            pltpu.VMEM((KB, 16), F32),
            pltpu.VMEM((KB, 16), F32),
            pltpu.SemaphoreType.DMA,
            pltpu.SemaphoreType.DMA,
        ],
    )(s_tab, d_tab, srcr, dstr, w_b, zeros_n2)


def _node_body(layer, last, acc0_ref, acc1_ref, c0_ref, pwr_ref, pwrf_ref,
               nl_ref, ap_ref, wnf_ref, wrws_ref, bws_ref, ws2_ref, bs2_ref,
               wrwp_ref, bwp_ref, *out_refs):
    a = acc0_ref[...] + acc1_ref[...]
    nf1 = a[:, 0:2]
    inv = 1.0 / jnp.maximum(a[:, 8:9], 1.0)
    nf2 = a[:, 2:8] * inv
    cp = c0_ref[...]
    for t in range(2):
        cp = cp + nf1[:, t:t + 1] * wnf_ref[t:t + 1, :]
    for t in range(6):
        cp = cp + nf2[:, t:t + 1] * wnf_ref[2 + t:3 + t, :]
    hc = _leaky(cp)
    hs = jnp.zeros_like(a[:, 0:4]) + bws_ref[...]
    for t in range(8):
        hs = hs + hc[:, t:t + 1] * wrws_ref[t:t + 1, :]
    hs = _leaky(hs)
    res = jnp.zeros_like(a[:, 0:2]) + bs2_ref[...]
    for t in range(4):
        res = res + hs[:, t:t + 1] * ws2_ref[t:t + 1, :]
    sc = 0.01 / float(layer ** 10)
    f0 = 0.95 + 0.1 / (1.0 + jnp.exp(-(res[:, 0:1] * sc)))
    f1 = 0.95 + 0.1 / (1.0 + jnp.exp(-(res[:, 1:2] * sc)))
    psum = a[:, 9:11]
    pfsum = a[:, 11:15]
    bn = a.shape[0]
    li2 = lax.broadcasted_iota(jnp.int32, (bn, 2), 1)
    li4 = lax.broadcasted_iota(jnp.int32, (bn, 4), 1)
    npwr = jnp.where(li2 == 0, psum * f0, psum)
    npwrf = jnp.where(li4 == 2, pfsum * f1, pfsum)
    nmask = nl_ref[...] == float(layer)
    out_refs[0][...] = jnp.where(nmask, npwr, pwr_ref[...])
    out_refs[1][...] = jnp.where(nmask, npwrf, pwrf_ref[...])
    if not last:
        anew = jnp.zeros_like(a[:, 0:8]) + bwp_ref[...]
        for t in range(8):
            anew = anew + hc[:, t:t + 1] * wrwp_ref[t:t + 1, :]
        out_refs[2][...] = jnp.where(nmask, anew, ap_ref[...])
        out_refs[3][...] = jnp.where(nl_ref[...] == float(layer + 1),
                                     1.0, 0.0).astype(F32)


def _node_call(layer, last, acc0, acc1, c0, pwr, pwrf, nl_f, a_prev,
               wnf, wrws, bwsr, ws2p, bs2r, wrwp, bwpr, bn):
    n = c0.shape[0]
    grid = (n // bn,)
    io = lambda i: (i, 0)
    w0 = lambda i: (0, 0)
    out_specs = [pl.BlockSpec((bn, 2), io), pl.BlockSpec((bn, 4), io)]
    out_shape = [jax.ShapeDtypeStruct((n, 2), F32),
                 jax.ShapeDtypeStruct((n, 4), F32)]
    if not last:
        out_specs += [pl.BlockSpec((bn, 8), io), pl.BlockSpec((bn, 1), io)]
        out_shape += [jax.ShapeDtypeStruct((n, 8), F32),
                      jax.ShapeDtypeStruct((n, 1), F32)]
    return pl.pallas_call(
        functools.partial(_node_body, layer, last),
        grid=grid,
        in_specs=[
            pl.BlockSpec((bn, 16), io),
            pl.BlockSpec((bn, 16), io),
            pl.BlockSpec((bn, 8), io),
            pl.BlockSpec((bn, 2), io),
            pl.BlockSpec((bn, 4), io),
            pl.BlockSpec((bn, 1), io),
            pl.BlockSpec((bn, 8), io),
            pl.BlockSpec((8, 8), w0),
            pl.BlockSpec((8, 4), w0),
            pl.BlockSpec((1, 4), w0),
            pl.BlockSpec((8, 2), w0),
            pl.BlockSpec((1, 2), w0),
            pl.BlockSpec((8, 8), w0),
            pl.BlockSpec((1, 8), w0),
        ],
        out_specs=out_specs,
        out_shape=out_shape,
    )(acc0, acc1, c0, pwr, pwrf, nl_f, a_prev,
      wnf, wrws, bwsr, ws2p, bs2r, wrwp, bwpr)


def kernel(nf, pwr, pwr_feat, edge_index, node_level, Wp1, bp1, Wp2, bp2,
           Wr1, br1, Wr2, br2, Ws1, bs1, Ws2, bs2):
    N = nf.shape[0]
    E = edge_index.shape[1]
    n2 = ((N + 1 + 127) // 128) * 128
    e2 = ((E + NW * KB - 1) // (NW * KB)) * (NW * KB)
    nblk = e2 // (NW * KB)
    bn = 1000 if N % 1000 == 0 else 8

    nl_f = node_level.astype(F32)[:, None]
    src = edge_index[0]
    dst = edge_index[1]
    if e2 != E:
        src = jnp.concatenate([src, jnp.zeros((e2 - E,), jnp.int32)])
        dst = jnp.concatenate([dst, jnp.full((e2 - E,), N, jnp.int32)])
    srcr = src.reshape(e2 // 128, 128)
    dstr = dst.reshape(e2 // 128, 128)

    wpt, wpb = Wp1[:128], Wp1[128:]
    wrt = Wr1[:128]
    wnf = Wr1[128:136]
    wrws = Wr2 @ Ws1
    bws = br2 @ Ws1 + bs1
    wrwp = Wr2 @ wpt
    bwp = br2 @ wpt
    ws2p = jnp.concatenate([Ws2, jnp.zeros((4, 2), F32)], axis=0)
    w_flat = jnp.concatenate([Wp2.reshape(-1), bp2,
                              jnp.zeros((96 - 81,), F32)])
    w_b = jnp.broadcast_to(w_flat[:, None], (96, 16)).astype(F32)
    zeros_n2 = jnp.zeros((n2, 16), F32)
    zrow = jnp.zeros((N, 1), F32)

    a_c, b_c, c0, em_c = _init_call(nf, wpt, wpb, bp1[None, :], wrt,
                                    br1[None, :], nl_f, bn)

    pwr_c, pwrf_c = pwr, pwr_feat
    for layer in (1, 2, 3):
        s_tab = jnp.concatenate([a_c, pwr_c, pwrf_c, zrow, zrow], axis=1)
        s_tab = jnp.concatenate(
            [s_tab, jnp.zeros((n2 - N, 16), F32)], axis=0)
        d_tab = jnp.concatenate(
            [b_c, em_c, jnp.broadcast_to(zrow, (N, 7))], axis=1)
        d_tab = jnp.concatenate(
            [d_tab, jnp.zeros((n2 - N, 16), F32)], axis=0)
        out = _edge_call(s_tab, d_tab, srcr, dstr, w_b, zeros_n2, n2, nblk)
        acc0 = out[0, :N]
        acc1 = out[1, :N]
        last = layer == 3
        r = _node_call(layer, last, acc0, acc1, c0, pwr_c, pwrf_c, nl_f, a_c,
                       wnf, wrws, bws[None, :], ws2p, bs2[None, :],
                       wrwp, bwp[None, :], bn)
        if last:
            pwr_c, pwrf_c = r
        else:
            pwr_c, pwrf_c, a_c, em_c = r
    return pwr_c, pwrf_c
